# baseline (speedup 1.0000x reference)
import jax
import jax.numpy as jnp
from jax.experimental import pallas as pl
from jax.experimental.pallas import tpu as pltpu

_BI = 400


def _support_body(x_ref, w_ref, out_ref):
    out_ref[...] = jnp.dot(x_ref[...], w_ref[...],
                           preferred_element_type=jnp.float32)


def _spmm_body(adj_ref, sup_ref, b_ref, out_ref):
    out_ref[...] = jnp.dot(adj_ref[...], sup_ref[...],
                           preferred_element_type=jnp.float32) + b_ref[...]


def kernel(input, adj, W, b):
    n, _ = input.shape
    d_out = W.shape[1]

    sup = pl.pallas_call(
        _support_body,
        out_shape=jax.ShapeDtypeStruct((n, d_out), jnp.float32),
    )(input, W)

    b2 = b.reshape(1, d_out)
    num_i = n // _BI
    out = pl.pallas_call(
        _spmm_body,
        grid=(num_i,),
        in_specs=[
            pl.BlockSpec((_BI, n), lambda i: (i, 0)),
            pl.BlockSpec((n, d_out), lambda i: (0, 0)),
            pl.BlockSpec((1, d_out), lambda i: (0, 0)),
        ],
        out_specs=pl.BlockSpec((_BI, d_out), lambda i: (i, 0)),
        out_shape=jax.ShapeDtypeStruct((n, d_out), jnp.float32),
        compiler_params=pltpu.CompilerParams(
            dimension_semantics=("parallel",)),
    )(adj, sup, b2)
    return out


# speedup vs baseline: 1.0509x; 1.0509x over previous
import jax
import jax.numpy as jnp
from jax.experimental import pallas as pl
from jax.experimental.pallas import tpu as pltpu

_BI = 400


def _fused_body(x_ref, w_ref, adj_ref, b_ref, out_ref, sup_ref):
    @pl.when(pl.program_id(0) == 0)
    def _():
        sup_ref[...] = jnp.dot(x_ref[...], w_ref[...],
                               preferred_element_type=jnp.float32)

    out_ref[...] = jnp.dot(adj_ref[...], sup_ref[...],
                           preferred_element_type=jnp.float32) + b_ref[...]


def kernel(input, adj, W, b):
    n, d_in = input.shape
    d_out = W.shape[1]
    b2 = b.reshape(1, d_out)
    num_i = n // _BI
    out = pl.pallas_call(
        _fused_body,
        grid=(num_i,),
        in_specs=[
            pl.BlockSpec((n, d_in), lambda i: (0, 0)),
            pl.BlockSpec((d_in, d_out), lambda i: (0, 0)),
            pl.BlockSpec((_BI, n), lambda i: (i, 0)),
            pl.BlockSpec((1, d_out), lambda i: (0, 0)),
        ],
        out_specs=pl.BlockSpec((_BI, d_out), lambda i: (i, 0)),
        out_shape=jax.ShapeDtypeStruct((n, d_out), jnp.float32),
        scratch_shapes=[pltpu.VMEM((n, d_out), jnp.float32)],
        compiler_params=pltpu.CompilerParams(
            dimension_semantics=("arbitrary",)),
    )(input, W, adj, b2)
    return out
